# baseline (speedup 1.0000x reference)
import jax
import jax.numpy as jnp
from jax.experimental import pallas as pl
from jax.experimental.pallas import tpu as pltpu

_B = 16384
_CHUNK = 1024
_NCHUNK = _B // _CHUNK
_NBUF = 4
_VIEW = (128, 256, 512)
_NW = 18


def _dot(a, b):
    return jnp.dot(a, b, preferred_element_type=jnp.float32)


def _body(*refs):
    xs = refs[0:3]
    wrefs = refs[3:3 + 3 * _NW]
    outs = refs[3 + 3 * _NW:12 + 3 * _NW]
    (xb0, xb1, xb2, rxb0, rxb1, rxb2, zcb0, zcb1, zcb2, zsb0, zsb1, zsb2,
     sin, srx, szc, szs) = refs[12 + 3 * _NW:]
    xbufs = (xb0, xb1, xb2)
    rxbufs = (rxb0, rxb1, rxb2)
    zcbufs = (zcb0, zcb1, zcb2)
    zsbufs = (zsb0, zsb1, zsb2)

    def in_copy(i):
        slot = i % _NBUF
        return [pltpu.make_async_copy(
            xs[v].at[pl.ds(i * _CHUNK, _CHUNK), :], xbufs[v].at[slot], sin.at[slot, v])
            for v in range(3)]

    def out_copy(i):
        slot = i % _NBUF
        cps = [pltpu.make_async_copy(
            rxbufs[v].at[slot], outs[6 + v].at[pl.ds(i * _CHUNK, _CHUNK), :], srx.at[slot, v])
            for v in range(3)]
        cps += [pltpu.make_async_copy(
            zcbufs[v].at[slot], outs[v].at[pl.ds(i * _CHUNK, _CHUNK), :], szc.at[slot, v])
            for v in range(3)]
        cps += [pltpu.make_async_copy(
            zsbufs[v].at[slot], outs[3 + v].at[pl.ds(i * _CHUNK, _CHUNK), :], szs.at[slot, v])
            for v in range(3)]
        return cps

    def compute(i):
        slot = i % _NBUF
        for v in range(3):
            (Wt, bt, Wc1, bc1, Wc2, bc2, Ws1, bs1, Ws2, bs2,
             Wdc, bdc, Wds, bds, Wd1, bd1, Wd2, bd2) = wrefs[v * _NW:(v + 1) * _NW]
            x = xbufs[v][slot]
            z1 = _dot(x, Wt[...]) + bt[...]
            hc = jnp.maximum(_dot(z1, Wc1[...]) + bc1[...], 0.0)
            zc = _dot(hc, Wc2[...]) + bc2[...]
            hs = jnp.maximum(_dot(z1, Ws1[...]) + bs1[...], 0.0)
            zs = _dot(hs, Ws2[...]) + bs2[...]
            dc = _dot(zc, Wdc[...]) + bdc[...]
            ds = _dot(zs, Wds[...]) + bds[...]
            d = jnp.concatenate([dc, ds], axis=1)
            g = jnp.maximum(_dot(d, Wd1[...]) + bd1[...], 0.0)
            rx = _dot(g, Wd2[...]) + bd2[...]
            zcbufs[v][slot] = zc
            zsbufs[v][slot] = zs
            rxbufs[v][slot] = rx

    for i in range(min(_NBUF, _NCHUNK)):
        for c in in_copy(i):
            c.start()
    for i in range(_NCHUNK):
        for c in in_copy(i):
            c.wait()
        if i >= _NBUF:
            for c in out_copy(i - _NBUF):
                c.wait()
        compute(i)
        for c in out_copy(i):
            c.start()
        if i + _NBUF < _NCHUNK:
            for c in in_copy(i + _NBUF):
                c.start()
    for i in range(max(_NCHUNK - _NBUF, 0), _NCHUNK):
        for c in out_copy(i):
            c.wait()


def kernel(x0, x1, x2, trunk_params, content_params, style_params,
           dec_content_params, dec_style_params, dec_trunk_params, status=0):
    xs = (x0, x1, x2)
    weights = []
    for v in range(3):
        (Wt, bt), = trunk_params[v]
        (Wc1, bc1), (Wc2, bc2) = content_params[v]
        (Ws1, bs1), (Ws2, bs2) = style_params[v]
        (Wdc, bdc), = dec_content_params[v]
        (Wds, bds), = dec_style_params[v]
        (Wd1, bd1), (Wd2, bd2) = dec_trunk_params[v]
        weights += [Wt, bt.reshape(1, -1), Wc1, bc1.reshape(1, -1),
                    Wc2, bc2.reshape(1, -1), Ws1, bs1.reshape(1, -1),
                    Ws2, bs2.reshape(1, -1), Wdc, bdc.reshape(1, -1),
                    Wds, bds.reshape(1, -1), Wd1, bd1.reshape(1, -1),
                    Wd2, bd2.reshape(1, -1)]

    any_spec = pl.BlockSpec(memory_space=pl.ANY)
    vmem = pl.BlockSpec(memory_space=pltpu.MemorySpace.VMEM)
    out_shape = (
        [jax.ShapeDtypeStruct((_B, 32), jnp.float32) for _ in range(6)]
        + [jax.ShapeDtypeStruct((_B, _VIEW[v]), jnp.float32) for v in range(3)]
    )
    scratch = (
        [pltpu.VMEM((_NBUF, _CHUNK, _VIEW[v]), jnp.float32) for v in range(3)]
        + [pltpu.VMEM((_NBUF, _CHUNK, _VIEW[v]), jnp.float32) for v in range(3)]
        + [pltpu.VMEM((_NBUF, _CHUNK, 32), jnp.float32) for _ in range(6)]
        + [pltpu.SemaphoreType.DMA((_NBUF, 3))] * 4
    )
    outs = pl.pallas_call(
        _body,
        in_specs=[any_spec] * 3 + [vmem] * len(weights),
        out_specs=[any_spec] * 9,
        out_shape=out_shape,
        scratch_shapes=scratch,
    )(*xs, *weights)
    return tuple(outs)
